# baseline (speedup 1.0000x reference)
import functools
import math

import jax
import jax.numpy as jnp
from jax import lax
from jax.experimental import pallas as pl
from jax.experimental.pallas import tpu as pltpu
from jax.experimental.pallas import tpu_sc as plsc

N = 10000
C = 128
E = 320000
NUM_LAYERS = 3
ALPHA = 0.1
THETA = 0.5

NC, NS = 2, 16
NW = NC * NS
K = 128
G = 79
EPT = G * K
E_PAD = NW * EPT
ROWS_PAD = 10240
RPT = ROWS_PAD // NS

BM = 1000


@functools.cache
def _sc_kernels():
    mesh = plsc.VectorSubcoreMesh(core_axis_name="c", subcore_axis_name="s",
                                  num_cores=NC, num_subcores=NS)

    @functools.partial(
        pl.kernel,
        out_type=jax.ShapeDtypeStruct((NC, ROWS_PAD, C), jnp.float32),
        mesh=mesh,
        scratch_types=[
            pltpu.VMEM((K,), jnp.int32),
            pltpu.VMEM((K,), jnp.int32),
            pltpu.VMEM((K, C), jnp.float32),
            pltpu.VMEM_SHARED((ROWS_PAD, C), jnp.float32),
            pltpu.SemaphoreType.DMA,
        ],
    )
    def sc_scatter(row_hbm, col_hbm, y_hbm, zeros_hbm, z_out,
                   ridx, cidx, rows, shared, sem):
        c = lax.axis_index("c")
        s = lax.axis_index("s")
        wid = c * NS + s
        pltpu.sync_copy(zeros_hbm, shared.at[pl.ds(s * RPT, RPT)])
        plsc.subcore_barrier()
        base = wid * EPT

        def body(g, carry):
            off = base + g * K
            pltpu.sync_copy(row_hbm.at[pl.ds(off, K)], ridx)
            pltpu.async_copy(y_hbm.at[ridx], rows, sem).wait()
            pltpu.sync_copy(col_hbm.at[pl.ds(off, K)], cidx)
            pltpu.sync_copy(rows, shared.at[cidx], add=True)
            return carry

        lax.fori_loop(0, G, body, 0)
        plsc.subcore_barrier()
        pltpu.sync_copy(shared.at[pl.ds(s * RPT, RPT)],
                        z_out.at[c, pl.ds(s * RPT, RPT)])

    return sc_scatter


def _fc1_body(nf, w, b, degw, x0_ref, y0_ref, dinv_ref):
    deg = degw[0, :, 0:1] + degw[1, :, 0:1] + 1.0
    dinv = lax.rsqrt(deg)
    x = jnp.maximum(
        jax.lax.dot_general(nf[...], w[...], (((1,), (0,)), ((), ())),
                            precision=lax.Precision.HIGHEST,
                            preferred_element_type=jnp.float32) + b[...], 0.0)
    x0_ref[...] = x
    y0_ref[...] = x * dinv
    dinv_ref[...] = jnp.broadcast_to(dinv, (BM, 16))


def _fc1(nf, w, b, degw):
    grid = (N // BM,)
    return pl.pallas_call(
        _fc1_body,
        grid=grid,
        in_specs=[
            pl.BlockSpec((BM, C), lambda i: (i, 0)),
            pl.BlockSpec((C, C), lambda i: (0, 0)),
            pl.BlockSpec((1, C), lambda i: (0, 0)),
            pl.BlockSpec((NC, BM, C), lambda i: (0, i, 0)),
        ],
        out_specs=[
            pl.BlockSpec((BM, C), lambda i: (i, 0)),
            pl.BlockSpec((BM, C), lambda i: (i, 0)),
            pl.BlockSpec((BM, 16), lambda i: (i, 0)),
        ],
        out_shape=[
            jax.ShapeDtypeStruct((N, C), jnp.float32),
            jax.ShapeDtypeStruct((N, C), jnp.float32),
            jax.ShapeDtypeStruct((N, 16), jnp.float32),
        ],
    )(nf, w, b, degw)


def _layer_body(z, y, x0, dinv, wp, y_ref):
    h = (z[0] + z[1] + y[...]) * dinv[:, 0:1]
    h = (1.0 - ALPHA) * h + ALPHA * x0[...]
    x = jnp.maximum(
        jax.lax.dot_general(h, wp[...], (((1,), (0,)), ((), ())),
                            precision=lax.Precision.HIGHEST,
                            preferred_element_type=jnp.float32), 0.0)
    y_ref[...] = x * dinv[:, 0:1]


def _layer(z, y, x0, dinv, wp):
    grid = (N // BM,)
    return pl.pallas_call(
        _layer_body,
        grid=grid,
        in_specs=[
            pl.BlockSpec((NC, BM, C), lambda i: (0, i, 0)),
            pl.BlockSpec((BM, C), lambda i: (i, 0)),
            pl.BlockSpec((BM, C), lambda i: (i, 0)),
            pl.BlockSpec((BM, 16), lambda i: (i, 0)),
            pl.BlockSpec((C, C), lambda i: (0, 0)),
        ],
        out_specs=pl.BlockSpec((BM, C), lambda i: (i, 0)),
        out_shape=jax.ShapeDtypeStruct((N, C), jnp.float32),
    )(z, y, x0, dinv, wp)


def _final_body(z, y, x0, dinv, wp, w2, b2, out_ref):
    h = (z[0] + z[1] + y[...]) * dinv[:, 0:1]
    h = (1.0 - ALPHA) * h + ALPHA * x0[...]
    x = jnp.maximum(
        jax.lax.dot_general(h, wp[...], (((1,), (0,)), ((), ())),
                            precision=lax.Precision.HIGHEST,
                            preferred_element_type=jnp.float32), 0.0)
    out_ref[...] = jax.lax.dot_general(
        x, w2[...], (((1,), (0,)), ((), ())),
        precision=lax.Precision.HIGHEST,
        preferred_element_type=jnp.float32) + b2[...]


def _final(z, y, x0, dinv, wp, w2, b2):
    grid = (N // BM,)
    return pl.pallas_call(
        _final_body,
        grid=grid,
        in_specs=[
            pl.BlockSpec((NC, BM, C), lambda i: (0, i, 0)),
            pl.BlockSpec((BM, C), lambda i: (i, 0)),
            pl.BlockSpec((BM, C), lambda i: (i, 0)),
            pl.BlockSpec((BM, 16), lambda i: (i, 0)),
            pl.BlockSpec((C, C), lambda i: (0, 0)),
            pl.BlockSpec((C, C), lambda i: (0, 0)),
            pl.BlockSpec((1, C), lambda i: (0, 0)),
        ],
        out_specs=pl.BlockSpec((BM, C), lambda i: (i, 0)),
        out_shape=jax.ShapeDtypeStruct((N, C), jnp.float32),
    )(z, y, x0, dinv, wp, w2, b2)


def kernel(node_features, edges, fc1_w, fc1_b, w1, w2, w3, fc2_w, fc2_b):
    row = jnp.concatenate(
        [edges[0].astype(jnp.int32), jnp.zeros((E_PAD - E,), jnp.int32)])
    col = jnp.concatenate(
        [edges[1].astype(jnp.int32),
         jnp.full((E_PAD - E,), N, jnp.int32)])
    eye = jnp.eye(C, dtype=jnp.float32)
    wps = []
    for i, w in enumerate((w1, w2, w3)):
        beta = math.log(THETA / (i + 1) + 1.0)
        wps.append(beta * w + (1.0 - beta) * eye)
    zeros_c = jnp.zeros((RPT, C), jnp.float32)
    zero_idx = jnp.zeros((E_PAD,), jnp.int32)
    ones_row = jnp.ones((8, C), jnp.float32)
    b1 = fc1_b.reshape(1, C)
    b2 = fc2_b.reshape(1, C)

    sc_scatter = _sc_kernels()
    degz = sc_scatter(zero_idx, col, ones_row, zeros_c)
    x0, y, dinv = _fc1(node_features, fc1_w, b1, degz)
    for i in range(NUM_LAYERS):
        z = sc_scatter(row, col, y, zeros_c)
        if i < NUM_LAYERS - 1:
            y = _layer(z, y, x0, dinv, wps[i])
        else:
            return _final(z, y, x0, dinv, wps[i], fc2_w, b2)


# speedup vs baseline: 9.4606x; 9.4606x over previous
import functools
import math

import jax
import jax.numpy as jnp
from jax import lax
from jax.experimental import pallas as pl
from jax.experimental.pallas import tpu as pltpu
from jax.experimental.pallas import tpu_sc as plsc

N = 10000
C = 128
E = 320000
NUM_LAYERS = 3
ALPHA = 0.1
THETA = 0.5

NC, NS = 2, 16
NW = NC * NS
K = 128
G = 79
EPT = G * K
E_PAD = NW * EPT
ROWS_PAD = 10240
RPT = ROWS_PAD // NS

BM = 1000


@functools.cache
def _sc_kernels():
    mesh = plsc.VectorSubcoreMesh(core_axis_name="c", subcore_axis_name="s",
                                  num_cores=NC, num_subcores=NS)

    @functools.partial(
        pl.kernel,
        out_type=jax.ShapeDtypeStruct((NC, ROWS_PAD, C), jnp.float32),
        mesh=mesh,
        scratch_types=[
            pltpu.VMEM((K,), jnp.int32),
            pltpu.VMEM((K,), jnp.int32),
            pltpu.VMEM((K, C), jnp.float32),
            pltpu.VMEM_SHARED((ROWS_PAD, C), jnp.float32),
            pltpu.SemaphoreType.DMA,
        ],
    )
    def sc_scatter(row_hbm, col_hbm, y_hbm, zeros_hbm, z_out,
                   ridx, cidx, rows, shared, sem):
        c = lax.axis_index("c")
        s = lax.axis_index("s")
        wid = c * NS + s
        pltpu.sync_copy(zeros_hbm, shared.at[pl.ds(s * RPT, RPT)])
        plsc.subcore_barrier()
        base = wid * EPT

        def body(g, carry):
            off = base + g * K
            pltpu.sync_copy(row_hbm.at[pl.ds(off, K)], ridx)
            pltpu.async_copy(y_hbm.at[ridx], rows, sem).wait()
            pltpu.sync_copy(col_hbm.at[pl.ds(off, K)], cidx)
            pltpu.sync_copy(rows, shared.at[cidx], add=True)
            return carry

        lax.fori_loop(0, G, body, 0)
        plsc.subcore_barrier()
        pltpu.sync_copy(shared.at[pl.ds(s * RPT, RPT)],
                        z_out.at[c, pl.ds(s * RPT, RPT)])

    @functools.partial(
        pl.kernel,
        out_type=jax.ShapeDtypeStruct((NC, ROWS_PAD, C), jnp.float32),
        mesh=mesh,
        scratch_types=[
            pltpu.VMEM((K,), jnp.int32),
            pltpu.VMEM((K, C), jnp.float32),
            pltpu.VMEM_SHARED((ROWS_PAD, C), jnp.float32),
        ],
    )
    def sc_degree(col_hbm, ones_hbm, zeros_hbm, z_out, cidx, ones_v, shared):
        c = lax.axis_index("c")
        s = lax.axis_index("s")
        wid = c * NS + s
        pltpu.sync_copy(zeros_hbm, shared.at[pl.ds(s * RPT, RPT)])
        pltpu.sync_copy(ones_hbm, ones_v)
        plsc.subcore_barrier()
        base = wid * EPT

        def body(g, carry):
            pltpu.sync_copy(col_hbm.at[pl.ds(base + g * K, K)], cidx)
            pltpu.sync_copy(ones_v, shared.at[cidx], add=True)
            return carry

        lax.fori_loop(0, G, body, 0)
        plsc.subcore_barrier()
        pltpu.sync_copy(shared.at[pl.ds(s * RPT, RPT)],
                        z_out.at[c, pl.ds(s * RPT, RPT)])

    return sc_degree, sc_scatter


def _fc1_body(nf, w, b, degw, x0_ref, y0_ref, dinv_ref):
    deg = degw[0, :, 0:1] + degw[1, :, 0:1] + 1.0
    dinv = lax.rsqrt(deg)
    x = jnp.maximum(
        jax.lax.dot_general(nf[...], w[...], (((1,), (0,)), ((), ())),
                            precision=lax.Precision.HIGHEST,
                            preferred_element_type=jnp.float32) + b[...], 0.0)
    x0_ref[...] = x
    y0_ref[...] = x * dinv
    dinv_ref[...] = jnp.broadcast_to(dinv, (BM, 16))


def _fc1(nf, w, b, degw):
    grid = (N // BM,)
    return pl.pallas_call(
        _fc1_body,
        grid=grid,
        in_specs=[
            pl.BlockSpec((BM, C), lambda i: (i, 0)),
            pl.BlockSpec((C, C), lambda i: (0, 0)),
            pl.BlockSpec((1, C), lambda i: (0, 0)),
            pl.BlockSpec((NC, BM, C), lambda i: (0, i, 0)),
        ],
        out_specs=[
            pl.BlockSpec((BM, C), lambda i: (i, 0)),
            pl.BlockSpec((BM, C), lambda i: (i, 0)),
            pl.BlockSpec((BM, 16), lambda i: (i, 0)),
        ],
        out_shape=[
            jax.ShapeDtypeStruct((N, C), jnp.float32),
            jax.ShapeDtypeStruct((N, C), jnp.float32),
            jax.ShapeDtypeStruct((N, 16), jnp.float32),
        ],
    )(nf, w, b, degw)


def _layer_body(z, y, x0, dinv, wp, y_ref):
    h = (z[0] + z[1] + y[...]) * dinv[:, 0:1]
    h = (1.0 - ALPHA) * h + ALPHA * x0[...]
    x = jnp.maximum(
        jax.lax.dot_general(h, wp[...], (((1,), (0,)), ((), ())),
                            precision=lax.Precision.HIGHEST,
                            preferred_element_type=jnp.float32), 0.0)
    y_ref[...] = x * dinv[:, 0:1]


def _layer(z, y, x0, dinv, wp):
    grid = (N // BM,)
    return pl.pallas_call(
        _layer_body,
        grid=grid,
        in_specs=[
            pl.BlockSpec((NC, BM, C), lambda i: (0, i, 0)),
            pl.BlockSpec((BM, C), lambda i: (i, 0)),
            pl.BlockSpec((BM, C), lambda i: (i, 0)),
            pl.BlockSpec((BM, 16), lambda i: (i, 0)),
            pl.BlockSpec((C, C), lambda i: (0, 0)),
        ],
        out_specs=pl.BlockSpec((BM, C), lambda i: (i, 0)),
        out_shape=jax.ShapeDtypeStruct((N, C), jnp.float32),
    )(z, y, x0, dinv, wp)


def _final_body(z, y, x0, dinv, wp, w2, b2, out_ref):
    h = (z[0] + z[1] + y[...]) * dinv[:, 0:1]
    h = (1.0 - ALPHA) * h + ALPHA * x0[...]
    x = jnp.maximum(
        jax.lax.dot_general(h, wp[...], (((1,), (0,)), ((), ())),
                            precision=lax.Precision.HIGHEST,
                            preferred_element_type=jnp.float32), 0.0)
    out_ref[...] = jax.lax.dot_general(
        x, w2[...], (((1,), (0,)), ((), ())),
        precision=lax.Precision.HIGHEST,
        preferred_element_type=jnp.float32) + b2[...]


def _final(z, y, x0, dinv, wp, w2, b2):
    grid = (N // BM,)
    return pl.pallas_call(
        _final_body,
        grid=grid,
        in_specs=[
            pl.BlockSpec((NC, BM, C), lambda i: (0, i, 0)),
            pl.BlockSpec((BM, C), lambda i: (i, 0)),
            pl.BlockSpec((BM, C), lambda i: (i, 0)),
            pl.BlockSpec((BM, 16), lambda i: (i, 0)),
            pl.BlockSpec((C, C), lambda i: (0, 0)),
            pl.BlockSpec((C, C), lambda i: (0, 0)),
            pl.BlockSpec((1, C), lambda i: (0, 0)),
        ],
        out_specs=pl.BlockSpec((BM, C), lambda i: (i, 0)),
        out_shape=jax.ShapeDtypeStruct((N, C), jnp.float32),
    )(z, y, x0, dinv, wp, w2, b2)


def kernel(node_features, edges, fc1_w, fc1_b, w1, w2, w3, fc2_w, fc2_b):
    row = jnp.concatenate(
        [edges[0].astype(jnp.int32), jnp.zeros((E_PAD - E,), jnp.int32)])
    col = jnp.concatenate(
        [edges[1].astype(jnp.int32),
         jnp.full((E_PAD - E,), N, jnp.int32)])
    eye = jnp.eye(C, dtype=jnp.float32)
    wps = []
    for i, w in enumerate((w1, w2, w3)):
        beta = math.log(THETA / (i + 1) + 1.0)
        wps.append(beta * w + (1.0 - beta) * eye)
    zeros_c = jnp.zeros((RPT, C), jnp.float32)
    ones_c = jnp.ones((K, C), jnp.float32)
    b1 = fc1_b.reshape(1, C)
    b2 = fc2_b.reshape(1, C)

    sc_degree, sc_scatter = _sc_kernels()
    degz = sc_degree(col, ones_c, zeros_c)
    x0, y, dinv = _fc1(node_features, fc1_w, b1, degz)
    for i in range(NUM_LAYERS):
        z = sc_scatter(row, col, y, zeros_c)
        if i < NUM_LAYERS - 1:
            y = _layer(z, y, x0, dinv, wps[i])
        else:
            return _final(z, y, x0, dinv, wps[i], fc2_w, b2)
